# baseline (speedup 1.0000x reference)
import functools

import jax
import jax.numpy as jnp
from jax import lax
from jax.experimental import pallas as pl
from jax.experimental.pallas import tpu as pltpu
from jax.experimental.pallas import tpu_sc as plsc

B, L, D = 4096, 200, 64
V = 100000
N = B * L
NC, NS = 2, 16
NW = NC * NS
STEP = 128
TPW = N // NW
NSTEPS = TPW // STEP
NBUF = 4


def _emb_body(tok_hbm, w0_hbm, w1_hbm, w2_hbm, out_hbm,
              idx_v, acc, sem_idx, sem_g0, sem_add, sem_out):
    cid = lax.axis_index("c")
    sid = lax.axis_index("s")
    wid = sid * NC + cid
    obase = wid * TPW
    ws = (w0_hbm, w1_hbm, w2_hbm)

    def drain_out(b):
        pltpu.make_async_copy(acc.at[b],
                              out_hbm.at[pl.ds(obase, STEP)],
                              sem_out.at[b]).wait()

    def fire(t, b):
        @pl.when(t >= NBUF)
        def _():
            drain_out(b)

        base = obase + t * STEP
        for j in range(3):
            pltpu.async_copy(tok_hbm.at[pl.ds(j * N + base, STEP)],
                             idx_v.at[b, j], sem_idx.at[b])
        for j in range(3):
            pltpu.make_async_copy(tok_hbm.at[pl.ds(j * N + base, STEP)],
                                  idx_v.at[b, j], sem_idx.at[b]).wait()
        pltpu.async_copy(ws[0].at[idx_v.at[b, 0]], acc.at[b], sem_g0.at[b])
        pltpu.make_async_copy(ws[0].at[idx_v.at[b, 0]], acc.at[b],
                              sem_g0.at[b]).wait()
        for j in (1, 2):
            pltpu.async_copy(ws[j].at[idx_v.at[b, j]], acc.at[b],
                             sem_add.at[b], add=True)

    def drain_add(b):
        for j in (1, 2):
            pltpu.make_async_copy(ws[j].at[idx_v.at[b, j]], acc.at[b],
                                  sem_add.at[b]).wait()

    def fire_out(t, b):
        pltpu.async_copy(acc.at[b],
                         out_hbm.at[pl.ds(obase + t * STEP, STEP)],
                         sem_out.at[b])

    fire(0, 0)

    def body4(u, carry):
        t0 = NBUF * u
        for b in range(NBUF):
            t = t0 + b

            @pl.when(t < NSTEPS - 1)
            def _():
                fire(t + 1, (b + 1) % NBUF)

            drain_add(b)
            fire_out(t, b)
        return carry

    lax.fori_loop(0, NSTEPS // NBUF, body4, 0)
    for b in range(NBUF):
        drain_out(b)


@jax.jit
def _emb_call(tokT, W0, W1, W2):
    mesh = plsc.VectorSubcoreMesh(core_axis_name="c", subcore_axis_name="s")
    return pl.kernel(
        _emb_body,
        out_type=jax.ShapeDtypeStruct((N, D), jnp.float32),
        mesh=mesh,
        scratch_types=[
            pltpu.VMEM((NBUF, 3, STEP), jnp.int32),
            pltpu.VMEM((NBUF, STEP, D), jnp.float32),
            pltpu.SemaphoreType.DMA((NBUF,)),
            pltpu.SemaphoreType.DMA((NBUF,)),
            pltpu.SemaphoreType.DMA((NBUF,)),
            pltpu.SemaphoreType.DMA((NBUF,)),
        ],
        compiler_params=pltpu.CompilerParams(use_tc_tiling_on_sc=False),
    )(tokT, W0, W1, W2)


def kernel(factored_tokens, W0, W1, W2):
    tokT = factored_tokens.reshape(N, 3).astype(jnp.int32).T.reshape(3 * N)
    out = _emb_call(tokT, W0, W1, W2)
    return out.reshape(B, L, D)


# speedup vs baseline: 1.0286x; 1.0286x over previous
import functools

import jax
import jax.numpy as jnp
from jax import lax
from jax.experimental import pallas as pl
from jax.experimental.pallas import tpu as pltpu
from jax.experimental.pallas import tpu_sc as plsc

B, L, D = 4096, 200, 64
V = 100000
N = B * L
NC, NS = 2, 16
NW = NC * NS
STEP = 128
TPW = N // NW
NSTEPS = TPW // STEP


def _emb_body(tok_hbm, w0_hbm, w1_hbm, w2_hbm, out_hbm,
              idx_v, rows, obuf, sem_idx, sem_in, sem_out):
    cid = lax.axis_index("c")
    sid = lax.axis_index("s")
    wid = sid * NC + cid
    obase = wid * TPW
    ws = (w0_hbm, w1_hbm, w2_hbm)

    def fire(t, b):
        base = obase + t * STEP
        for j in range(3):
            pltpu.async_copy(tok_hbm.at[pl.ds(j * N + base, STEP)],
                             idx_v.at[b, j], sem_idx.at[b])
        for j in range(3):
            pltpu.make_async_copy(tok_hbm.at[pl.ds(j * N + base, STEP)],
                                  idx_v.at[b, j], sem_idx.at[b]).wait()
        for j in range(3):
            pltpu.async_copy(ws[j].at[idx_v.at[b, j]], rows.at[b, j],
                             sem_in.at[b])

    def drain(b):
        for j in range(3):
            pltpu.make_async_copy(ws[j].at[idx_v.at[b, j]], rows.at[b, j],
                                  sem_in.at[b]).wait()

    def accum(b):
        @plsc.parallel_loop(0, STEP, unroll=4)
        def addrow(r):
            for k in range(D // 16):
                sl = pl.ds(k * 16, 16)
                obuf[b, r, sl] = (rows[b, 0, r, sl] + rows[b, 1, r, sl]
                                  + rows[b, 2, r, sl])

    def fire_out(t, b):
        pltpu.async_copy(obuf.at[b],
                         out_hbm.at[pl.ds(obase + t * STEP, STEP)],
                         sem_out.at[b])

    def drain_out(b):
        pltpu.make_async_copy(obuf.at[b],
                              out_hbm.at[pl.ds(obase, STEP)],
                              sem_out.at[b]).wait()

    fire(0, 0)

    def body2(u, carry):
        t0 = 2 * u
        for b in range(2):
            t = t0 + b

            @pl.when(t < NSTEPS - 1)
            def _():
                fire(t + 1, 1 - b)

            drain(b)

            @pl.when(t >= 2)
            def _():
                drain_out(b)

            accum(b)
            fire_out(t, b)
        return carry

    lax.fori_loop(0, NSTEPS // 2, body2, 0)
    drain_out(0)
    drain_out(1)


@jax.jit
def _emb_call(tokT, W0, W1, W2):
    mesh = plsc.VectorSubcoreMesh(core_axis_name="c", subcore_axis_name="s")
    return pl.kernel(
        _emb_body,
        out_type=jax.ShapeDtypeStruct((N, D), jnp.float32),
        mesh=mesh,
        scratch_types=[
            pltpu.VMEM((2, 3, STEP), jnp.int32),
            pltpu.VMEM((2, 3, STEP, D), jnp.float32),
            pltpu.VMEM((2, STEP, D), jnp.float32),
            pltpu.SemaphoreType.DMA((2,)),
            pltpu.SemaphoreType.DMA((2,)),
            pltpu.SemaphoreType.DMA((2,)),
        ],
        compiler_params=pltpu.CompilerParams(use_tc_tiling_on_sc=False),
    )(tokT, W0, W1, W2)


def kernel(factored_tokens, W0, W1, W2):
    tokT = factored_tokens.reshape(N, 3).astype(jnp.int32).T.reshape(3 * N)
    out = _emb_call(tokT, W0, W1, W2)
    return out.reshape(B, L, D)


# speedup vs baseline: 1.0729x; 1.0431x over previous
import functools

import jax
import jax.numpy as jnp
from jax import lax
from jax.experimental import pallas as pl
from jax.experimental.pallas import tpu as pltpu
from jax.experimental.pallas import tpu_sc as plsc

B, L, D = 4096, 200, 64
V = 100000
N = B * L
NC, NS = 2, 16
NW = NC * NS
STEP = 128
TPW = N // NW
NSTEPS = TPW // STEP
NI = 4


def _emb_body(tok_hbm, w0_hbm, w1_hbm, w2_hbm, out_hbm,
              idx_v, rows, obuf, sem_idx, sem_in, sem_out):
    cid = lax.axis_index("c")
    sid = lax.axis_index("s")
    wid = sid * NC + cid
    obase = wid * TPW
    ws = (w0_hbm, w1_hbm, w2_hbm)

    def fire_idx(t, bi):
        base = obase + t * STEP
        for j in range(3):
            pltpu.async_copy(tok_hbm.at[pl.ds(j * N + base, STEP)],
                             idx_v.at[bi, j], sem_idx.at[bi])

    def wait_idx(bi):
        for j in range(3):
            pltpu.make_async_copy(tok_hbm.at[pl.ds(obase, STEP)],
                                  idx_v.at[bi, j], sem_idx.at[bi]).wait()

    def fire_gather(b, bi):
        for j in range(3):
            pltpu.async_copy(ws[j].at[idx_v.at[bi, j]], rows.at[b, j],
                             sem_in.at[b])

    def drain_gather(b, bi):
        for j in range(3):
            pltpu.make_async_copy(ws[j].at[idx_v.at[bi, j]], rows.at[b, j],
                                  sem_in.at[b]).wait()

    def accum(b):
        @plsc.parallel_loop(0, STEP, unroll=4)
        def addrow(r):
            for k in range(D // 16):
                sl = pl.ds(k * 16, 16)
                obuf[b, r, sl] = (rows[b, 0, r, sl] + rows[b, 1, r, sl]
                                  + rows[b, 2, r, sl])

    def fire_out(t, b):
        pltpu.async_copy(obuf.at[b],
                         out_hbm.at[pl.ds(obase + t * STEP, STEP)],
                         sem_out.at[b])

    def drain_out(b):
        pltpu.make_async_copy(obuf.at[b],
                              out_hbm.at[pl.ds(obase, STEP)],
                              sem_out.at[b]).wait()

    for t in range(3):
        fire_idx(t, t)
    wait_idx(0)
    fire_gather(0, 0)

    def body4(u, carry):
        t0 = 4 * u
        for i in range(4):
            t = t0 + i
            b = i % 2
            bi = i

            @pl.when(t < NSTEPS - 3)
            def _():
                fire_idx(t + 3, (bi + 3) % NI)

            @pl.when(t < NSTEPS - 1)
            def _():
                wait_idx((bi + 1) % NI)
                fire_gather(1 - b, (bi + 1) % NI)

            drain_gather(b, bi)

            @pl.when(t >= 2)
            def _():
                drain_out(b)

            accum(b)
            fire_out(t, b)
        return carry

    lax.fori_loop(0, NSTEPS // 4, body4, 0)
    drain_out(0)
    drain_out(1)


@jax.jit
def _emb_call(tokT, W0, W1, W2):
    mesh = plsc.VectorSubcoreMesh(core_axis_name="c", subcore_axis_name="s")
    return pl.kernel(
        _emb_body,
        out_type=jax.ShapeDtypeStruct((N, D), jnp.float32),
        mesh=mesh,
        scratch_types=[
            pltpu.VMEM((NI, 3, STEP), jnp.int32),
            pltpu.VMEM((2, 3, STEP, D), jnp.float32),
            pltpu.VMEM((2, STEP, D), jnp.float32),
            pltpu.SemaphoreType.DMA((NI,)),
            pltpu.SemaphoreType.DMA((2,)),
            pltpu.SemaphoreType.DMA((2,)),
        ],
        compiler_params=pltpu.CompilerParams(use_tc_tiling_on_sc=False),
    )(tokT, W0, W1, W2)


def kernel(factored_tokens, W0, W1, W2):
    tokT = factored_tokens.reshape(N, 3).astype(jnp.int32).T.reshape(3 * N)
    out = _emb_call(tokT, W0, W1, W2)
    return out.reshape(B, L, D)


# speedup vs baseline: 1.0817x; 1.0082x over previous
import functools

import jax
import jax.numpy as jnp
from jax import lax
from jax.experimental import pallas as pl
from jax.experimental.pallas import tpu as pltpu
from jax.experimental.pallas import tpu_sc as plsc

B, L, D = 4096, 200, 64
V = 100000
N = B * L
NC, NS = 2, 16
NW = NC * NS
STEP = 160
TPW = N // NW
NSTEPS = TPW // STEP
NI = 4


def _emb_body(tok_hbm, w0_hbm, w1_hbm, w2_hbm, out_hbm,
              idx_v, rows, obuf, sem_idx, sem_in, sem_out):
    cid = lax.axis_index("c")
    sid = lax.axis_index("s")
    wid = sid * NC + cid
    obase = wid * TPW
    ws = (w0_hbm, w1_hbm, w2_hbm)

    def fire_idx(t, bi):
        base = obase + t * STEP
        for j in range(3):
            pltpu.async_copy(tok_hbm.at[pl.ds(j * N + base, STEP)],
                             idx_v.at[bi, j], sem_idx.at[bi])

    def wait_idx(bi):
        for j in range(3):
            pltpu.make_async_copy(tok_hbm.at[pl.ds(obase, STEP)],
                                  idx_v.at[bi, j], sem_idx.at[bi]).wait()

    def fire_gather(b, bi):
        for j in range(3):
            pltpu.async_copy(ws[j].at[idx_v.at[bi, j]], rows.at[b, j],
                             sem_in.at[b])

    def drain_gather(b, bi):
        for j in range(3):
            pltpu.make_async_copy(ws[j].at[idx_v.at[bi, j]], rows.at[b, j],
                                  sem_in.at[b]).wait()

    def accum(b):
        @plsc.parallel_loop(0, STEP, unroll=4)
        def addrow(r):
            for k in range(D // 16):
                sl = pl.ds(k * 16, 16)
                obuf[b, r, sl] = (rows[b, 0, r, sl] + rows[b, 1, r, sl]
                                  + rows[b, 2, r, sl])

    def fire_out(t, b):
        pltpu.async_copy(obuf.at[b],
                         out_hbm.at[pl.ds(obase + t * STEP, STEP)],
                         sem_out.at[b])

    def drain_out(b):
        pltpu.make_async_copy(obuf.at[b],
                              out_hbm.at[pl.ds(obase, STEP)],
                              sem_out.at[b]).wait()

    for t in range(3):
        fire_idx(t, t)
    wait_idx(0)
    fire_gather(0, 0)

    def body4(u, carry):
        t0 = 4 * u
        for i in range(4):
            t = t0 + i
            b = i % 2
            bi = i

            @pl.when(t < NSTEPS - 3)
            def _():
                fire_idx(t + 3, (bi + 3) % NI)

            @pl.when(t < NSTEPS - 1)
            def _():
                wait_idx((bi + 1) % NI)
                fire_gather(1 - b, (bi + 1) % NI)

            drain_gather(b, bi)

            @pl.when(t >= 2)
            def _():
                drain_out(b)

            accum(b)
            fire_out(t, b)
        return carry

    lax.fori_loop(0, NSTEPS // 4, body4, 0)
    drain_out(0)
    drain_out(1)


@jax.jit
def _emb_call(tokT, W0, W1, W2):
    mesh = plsc.VectorSubcoreMesh(core_axis_name="c", subcore_axis_name="s")
    return pl.kernel(
        _emb_body,
        out_type=jax.ShapeDtypeStruct((N, D), jnp.float32),
        mesh=mesh,
        scratch_types=[
            pltpu.VMEM((NI, 3, STEP), jnp.int32),
            pltpu.VMEM((2, 3, STEP, D), jnp.float32),
            pltpu.VMEM((2, STEP, D), jnp.float32),
            pltpu.SemaphoreType.DMA((NI,)),
            pltpu.SemaphoreType.DMA((2,)),
            pltpu.SemaphoreType.DMA((2,)),
        ],
        compiler_params=pltpu.CompilerParams(use_tc_tiling_on_sc=False),
    )(tokT, W0, W1, W2)


def kernel(factored_tokens, W0, W1, W2):
    tokT = factored_tokens.reshape(N, 3).astype(jnp.int32).T.reshape(3 * N)
    out = _emb_call(tokT, W0, W1, W2)
    return out.reshape(B, L, D)
